# baseline (speedup 1.0000x reference)
import math

import jax
import jax.numpy as jnp
from jax import lax
from jax.experimental import pallas as pl
from jax.experimental.pallas import tpu as pltpu
from jax.experimental.pallas import tpu_sc as plsc

N, E, D, R, T, G = 10000, 320000, 128, 4, 12, 256
BN_EPS = 1e-5
_BN_INV = 1.0 / math.sqrt(1.0 + BN_EPS)

NC, NS = 2, 16
NPAD = 10240
ZR = NPAD // NS
UCH = 64
UNITS = 320
EPT = UNITS * UCH
DUMMY_DST = N
TRASH = NPAD - 8
SRC_SHIFT = 16

PK = 3
PKP = 8
PCH = 128
NPOOL = NC * NS * PK * PCH
PR = 384
PZR = PR // NS

BL = 400


def _agg_body(h_hbm, packed_hbm, zrow_hbm, out_hbm,
              seg_sp, packed_v, sidxA, didxA, sidxB, didxB,
              rowsA, rowsB, semA, semB):
    cid = lax.axis_index("c")
    sid = lax.axis_index("s")
    pltpu.sync_copy(packed_hbm.at[pl.ds(sid * EPT, EPT)], packed_v)

    for p in range(2):
        for cc in range(NC):
            rr = 2 * p + cc

            @pl.when(cid == cc)
            def _(rr=rr):
                pltpu.sync_copy(zrow_hbm, seg_sp.at[pl.ds(sid * ZR, ZR)])
                plsc.subcore_barrier()

                def compute(u, sx, dx, rr=rr):
                    for v in range(UCH // 16):
                        w = packed_v[pl.ds(u * UCH + v * 16, 16)]
                        off = (w & 0xFFFF) - rr * NPAD
                        ok = (off >= 0) & (off < NPAD)
                        sx[0, pl.ds(v * 16, 16)] = jnp.where(
                            ok, lax.shift_right_logical(w, SRC_SHIFT), 0)
                        dx[0, pl.ds(v * 16, 16)] = jnp.where(ok, off, TRASH)

                compute(0, sidxA, didxA)
                pltpu.async_copy(h_hbm.at[sidxA.at[0]], rowsA, semA)

                def unit_body(j, carry):
                    compute(2 * j + 1, sidxB, didxB)
                    pltpu.make_async_copy(
                        h_hbm.at[sidxA.at[0]], rowsA, semA).wait()
                    pltpu.async_copy(h_hbm.at[sidxB.at[0]], rowsB, semB)
                    pltpu.sync_copy(rowsA, seg_sp.at[didxA.at[0]], add=True)
                    compute(2 * j + 2, sidxA, didxA)
                    pltpu.make_async_copy(
                        h_hbm.at[sidxB.at[0]], rowsB, semB).wait()
                    pltpu.async_copy(h_hbm.at[sidxA.at[0]], rowsA, semA)
                    pltpu.sync_copy(rowsB, seg_sp.at[didxB.at[0]], add=True)
                    return carry

                lax.fori_loop(0, UNITS // 2 - 1, unit_body, 0)
                compute(UNITS - 1, sidxB, didxB)
                pltpu.make_async_copy(
                    h_hbm.at[sidxA.at[0]], rowsA, semA).wait()
                pltpu.async_copy(h_hbm.at[sidxB.at[0]], rowsB, semB)
                pltpu.sync_copy(rowsA, seg_sp.at[didxA.at[0]], add=True)
                pltpu.make_async_copy(
                    h_hbm.at[sidxB.at[0]], rowsB, semB).wait()
                pltpu.sync_copy(rowsB, seg_sp.at[didxB.at[0]], add=True)

                plsc.subcore_barrier()
                pltpu.sync_copy(seg_sp.at[pl.ds(sid * ZR, ZR)],
                                out_hbm.at[rr].at[pl.ds(sid * ZR, ZR)])
                plsc.subcore_barrier()


_sc_mesh = plsc.VectorSubcoreMesh(
    core_axis_name="c", subcore_axis_name="s", num_cores=NC, num_subcores=NS)

_agg = pl.kernel(
    _agg_body,
    out_type=jax.ShapeDtypeStruct((R, NPAD, D), jnp.float32),
    mesh=_sc_mesh,
    scratch_types=[
        pltpu.VMEM_SHARED((NPAD, D), jnp.float32),
        pltpu.VMEM((EPT,), jnp.int32),
        pltpu.VMEM((1, UCH), jnp.int32),
        pltpu.VMEM((1, UCH), jnp.int32),
        pltpu.VMEM((1, UCH), jnp.int32),
        pltpu.VMEM((1, UCH), jnp.int32),
        pltpu.VMEM((UCH, D), jnp.float32),
        pltpu.VMEM((UCH, D), jnp.float32),
        pltpu.SemaphoreType.DMA,
        pltpu.SemaphoreType.DMA,
    ],
)


def _pool_body(h_hbm, ids_hbm, ones_hbm, zrow_hbm, sums_hbm, cnt_hbm,
               pool_sp, cnt_sp, ids_v, rows_v, ones_v):
    cid = lax.axis_index("c")
    sid = lax.axis_index("s")
    wid = cid * NS + sid
    pltpu.sync_copy(ids_hbm.at[wid], ids_v)
    pltpu.sync_copy(ones_hbm, ones_v)
    pltpu.sync_copy(zrow_hbm, pool_sp.at[pl.ds(sid * PZR, PZR)])
    pltpu.sync_copy(zrow_hbm, cnt_sp.at[pl.ds(sid * PZR, PZR)])
    plsc.subcore_barrier()
    base = wid * PK * PCH
    for k in range(PK):
        pltpu.sync_copy(h_hbm.at[pl.ds(base + k * PCH, PCH)], rows_v)
        pltpu.sync_copy(rows_v, pool_sp.at[ids_v.at[k]], add=True)
        pltpu.sync_copy(ones_v, cnt_sp.at[ids_v.at[k]], add=True)
    plsc.subcore_barrier()
    pltpu.sync_copy(pool_sp.at[pl.ds(sid * PZR, PZR)],
                    sums_hbm.at[cid].at[pl.ds(sid * PZR, PZR)])
    pltpu.sync_copy(cnt_sp.at[pl.ds(sid * PZR, PZR)],
                    cnt_hbm.at[cid].at[pl.ds(sid * PZR, PZR)])


_pool = pl.kernel(
    _pool_body,
    out_type=(jax.ShapeDtypeStruct((NC, PR, D), jnp.float32),
              jax.ShapeDtypeStruct((NC, PR, D), jnp.float32)),
    mesh=_sc_mesh,
    scratch_types=[
        pltpu.VMEM_SHARED((PR, D), jnp.float32),
        pltpu.VMEM_SHARED((PR, D), jnp.float32),
        pltpu.VMEM((PKP, PCH), jnp.int32),
        pltpu.VMEM((PCH, D), jnp.float32),
        pltpu.VMEM((PCH, D), jnp.float32),
    ],
)


def _layer_body(h_ref, agg_ref, slW_ref, slb_ref, W1_ref, b1_ref, g_ref,
                be_ref, W2_ref, b2_ref, out_ref):
    h = h_ref[...]
    acc = jnp.dot(h, slW_ref[...], preferred_element_type=jnp.float32)
    acc = acc + slb_ref[...]
    for r in range(R):
        t = h + agg_ref[r]
        u = jnp.dot(t, W1_ref[r], preferred_element_type=jnp.float32)
        u = (u + b1_ref[r]) * (g_ref[r] * _BN_INV) + be_ref[r]
        u = jnp.maximum(u, 0.0)
        acc = acc + jnp.dot(u, W2_ref[r], preferred_element_type=jnp.float32)
        acc = acc + b2_ref[r]
    out_ref[...] = jnp.maximum(acc, 0.0)


_layer = pl.pallas_call(
    _layer_body,
    grid=(N // BL,),
    in_specs=[
        pl.BlockSpec((BL, D), lambda i: (i, 0)),
        pl.BlockSpec((R, BL, D), lambda i: (0, i, 0)),
        pl.BlockSpec((D, D), lambda i: (0, 0)),
        pl.BlockSpec((1, D), lambda i: (0, 0)),
        pl.BlockSpec((R, D, D), lambda i: (0, 0, 0)),
        pl.BlockSpec((R, 1, D), lambda i: (0, 0, 0)),
        pl.BlockSpec((R, 1, D), lambda i: (0, 0, 0)),
        pl.BlockSpec((R, 1, D), lambda i: (0, 0, 0)),
        pl.BlockSpec((R, D, D), lambda i: (0, 0, 0)),
        pl.BlockSpec((R, 1, D), lambda i: (0, 0, 0)),
    ],
    out_specs=pl.BlockSpec((BL, D), lambda i: (i, 0)),
    out_shape=jax.ShapeDtypeStruct((N, D), jnp.float32),
)


def _head_body(sums_ref, cnt_ref, w_ref, b_ref, out_ref):
    s = sums_ref[0, :G, :] + sums_ref[1, :G, :]
    c = cnt_ref[0, :G, :1] + cnt_ref[1, :G, :1]
    pooled = s / jnp.maximum(c, 1.0)
    out_ref[...] = (jnp.dot(pooled, w_ref[...],
                            preferred_element_type=jnp.float32) + b_ref[...])


_head = pl.pallas_call(
    _head_body,
    in_specs=[
        pl.BlockSpec((NC, PR, D), lambda: (0, 0, 0)),
        pl.BlockSpec((NC, PR, D), lambda: (0, 0, 0)),
        pl.BlockSpec((D, T), lambda: (0, 0)),
        pl.BlockSpec((1, T), lambda: (0, 0)),
    ],
    out_specs=pl.BlockSpec((G, T), lambda: (0, 0)),
    out_shape=jax.ShapeDtypeStruct((G, T), jnp.float32),
)


def kernel(x, edge_index, edge_type, batch,
           c1_slW, c1_slb, c1_W1, c1_b1, c1_g, c1_be, c1_W2, c1_b2,
           c2_slW, c2_slb, c2_W1, c2_b1, c2_g, c2_be, c2_W2, c2_b2,
           c3_slW, c3_slb, c3_W1, c3_b1, c3_g, c3_be, c3_W2, c3_b2,
           lin_W, lin_b):
    src = edge_index[0]
    dst = edge_index[1]
    packed = jnp.left_shift(src, SRC_SHIFT) | (edge_type * NPAD + dst)
    packed = jnp.concatenate(
        [packed, jnp.full((NS * EPT - E,), DUMMY_DST, jnp.int32)])
    zrow = jnp.zeros((ZR, D), jnp.float32)

    h = x
    for (slW, slb, W1, b1, g, be, W2, b2) in (
            (c1_slW, c1_slb, c1_W1, c1_b1, c1_g, c1_be, c1_W2, c1_b2),
            (c2_slW, c2_slb, c2_W1, c2_b1, c2_g, c2_be, c2_W2, c2_b2),
            (c3_slW, c3_slb, c3_W1, c3_b1, c3_g, c3_be, c3_W2, c3_b2)):
        agg4 = _agg(h, packed, zrow)
        h = _layer(h, agg4, slW, slb.reshape(1, D), W1,
                   b1.reshape(R, 1, D), g.reshape(R, 1, D),
                   be.reshape(R, 1, D), W2, b2.reshape(R, 1, D))

    h_pad = jnp.concatenate([h, jnp.zeros((NPOOL - N, D), jnp.float32)])
    ids_p = jnp.concatenate(
        [batch, jnp.full((NPOOL - N,), G, jnp.int32)]).reshape(
            NC * NS, PK, PCH)
    ids_p = jnp.pad(ids_p, ((0, 0), (0, PKP - PK), (0, 0)),
                    constant_values=G)
    ones = jnp.ones((PCH, D), jnp.float32)
    zrow_pool = jnp.zeros((PZR, D), jnp.float32)
    sums, cnt = _pool(h_pad, ids_p, ones, zrow_pool)
    return _head(sums, cnt, lin_W, lin_b.reshape(1, T))


# speedup vs baseline: 22.0556x; 22.0556x over previous
import math

import jax
import jax.numpy as jnp
from jax import lax
from jax.experimental import pallas as pl
from jax.experimental.pallas import tpu as pltpu
from jax.experimental.pallas import tpu_sc as plsc

N, E, D, R, T, G = 10000, 320000, 128, 4, 12, 256
BN_EPS = 1e-5
_BN_INV = 1.0 / math.sqrt(1.0 + BN_EPS)

NC, NS = 2, 16
NPAD = 10240
ZR = NPAD // NS
UCH = 64
UNITS = 320
EPT = UNITS * UCH
DUMMY_DST = N
TRASH = NPAD - 64
SRC_SHIFT = 16

PK = 3
PKP = 8
PCH = 128
NPOOL = NC * NS * PK * PCH
PR = 384
PZR = PR // NS

BL = 400


def _agg_body(h_hbm, packed_hbm, zrow_hbm, out_hbm,
              seg_sp, packed_v, sidxA, didxA, sidxB, didxB,
              rowsA, rowsB, semA, semB):
    cid = lax.axis_index("c")
    sid = lax.axis_index("s")
    pltpu.sync_copy(packed_hbm.at[pl.ds(sid * EPT, EPT)], packed_v)

    for p in range(2):
        for cc in range(NC):
            rr = 2 * p + cc

            @pl.when(cid == cc)
            def _(rr=rr):
                pltpu.sync_copy(zrow_hbm, seg_sp.at[pl.ds(sid * ZR, ZR)])
                plsc.subcore_barrier()

                def compute(u, sx, dx, rr=rr):
                    iota = lax.iota(jnp.int32, 16)
                    for v in range(UCH // 16):
                        w = packed_v[pl.ds(u * UCH + v * 16, 16)]
                        off = (w & 0xFFFF) - rr * NPAD
                        ok = (off >= 0) & (off < NPAD)
                        sx[0, pl.ds(v * 16, 16)] = lax.shift_right_logical(
                            w, SRC_SHIFT)
                        dx[0, pl.ds(v * 16, 16)] = jnp.where(
                            ok, off, TRASH + v * 16 + iota)

                compute(0, sidxA, didxA)
                pltpu.async_copy(h_hbm.at[sidxA.at[0]], rowsA, semA)

                def unit_body(j, carry):
                    compute(2 * j + 1, sidxB, didxB)
                    pltpu.make_async_copy(
                        h_hbm.at[sidxA.at[0]], rowsA, semA).wait()
                    pltpu.async_copy(h_hbm.at[sidxB.at[0]], rowsB, semB)
                    pltpu.sync_copy(rowsA, seg_sp.at[didxA.at[0]], add=True)
                    compute(2 * j + 2, sidxA, didxA)
                    pltpu.make_async_copy(
                        h_hbm.at[sidxB.at[0]], rowsB, semB).wait()
                    pltpu.async_copy(h_hbm.at[sidxA.at[0]], rowsA, semA)
                    pltpu.sync_copy(rowsB, seg_sp.at[didxB.at[0]], add=True)
                    return carry

                lax.fori_loop(0, UNITS // 2 - 1, unit_body, 0)
                compute(UNITS - 1, sidxB, didxB)
                pltpu.make_async_copy(
                    h_hbm.at[sidxA.at[0]], rowsA, semA).wait()
                pltpu.async_copy(h_hbm.at[sidxB.at[0]], rowsB, semB)
                pltpu.sync_copy(rowsA, seg_sp.at[didxA.at[0]], add=True)
                pltpu.make_async_copy(
                    h_hbm.at[sidxB.at[0]], rowsB, semB).wait()
                pltpu.sync_copy(rowsB, seg_sp.at[didxB.at[0]], add=True)

                plsc.subcore_barrier()
                pltpu.sync_copy(seg_sp.at[pl.ds(sid * ZR, ZR)],
                                out_hbm.at[rr].at[pl.ds(sid * ZR, ZR)])
                plsc.subcore_barrier()


_sc_mesh = plsc.VectorSubcoreMesh(
    core_axis_name="c", subcore_axis_name="s", num_cores=NC, num_subcores=NS)

_agg = pl.kernel(
    _agg_body,
    out_type=jax.ShapeDtypeStruct((R, NPAD, D), jnp.float32),
    mesh=_sc_mesh,
    scratch_types=[
        pltpu.VMEM_SHARED((NPAD, D), jnp.float32),
        pltpu.VMEM((EPT,), jnp.int32),
        pltpu.VMEM((1, UCH), jnp.int32),
        pltpu.VMEM((1, UCH), jnp.int32),
        pltpu.VMEM((1, UCH), jnp.int32),
        pltpu.VMEM((1, UCH), jnp.int32),
        pltpu.VMEM((UCH, D), jnp.float32),
        pltpu.VMEM((UCH, D), jnp.float32),
        pltpu.SemaphoreType.DMA,
        pltpu.SemaphoreType.DMA,
    ],
)


def _pool_body(h_hbm, ids_hbm, ones_hbm, zrow_hbm, sums_hbm, cnt_hbm,
               pool_sp, cnt_sp, ids_v, rows_v, ones_v):
    cid = lax.axis_index("c")
    sid = lax.axis_index("s")
    wid = cid * NS + sid
    pltpu.sync_copy(ids_hbm.at[wid], ids_v)
    pltpu.sync_copy(ones_hbm, ones_v)
    pltpu.sync_copy(zrow_hbm, pool_sp.at[pl.ds(sid * PZR, PZR)])
    pltpu.sync_copy(zrow_hbm, cnt_sp.at[pl.ds(sid * PZR, PZR)])
    plsc.subcore_barrier()
    base = wid * PK * PCH
    for k in range(PK):
        pltpu.sync_copy(h_hbm.at[pl.ds(base + k * PCH, PCH)], rows_v)
        pltpu.sync_copy(rows_v, pool_sp.at[ids_v.at[k]], add=True)
        pltpu.sync_copy(ones_v, cnt_sp.at[ids_v.at[k]], add=True)
    plsc.subcore_barrier()
    pltpu.sync_copy(pool_sp.at[pl.ds(sid * PZR, PZR)],
                    sums_hbm.at[cid].at[pl.ds(sid * PZR, PZR)])
    pltpu.sync_copy(cnt_sp.at[pl.ds(sid * PZR, PZR)],
                    cnt_hbm.at[cid].at[pl.ds(sid * PZR, PZR)])


_pool = pl.kernel(
    _pool_body,
    out_type=(jax.ShapeDtypeStruct((NC, PR, D), jnp.float32),
              jax.ShapeDtypeStruct((NC, PR, D), jnp.float32)),
    mesh=_sc_mesh,
    scratch_types=[
        pltpu.VMEM_SHARED((PR, D), jnp.float32),
        pltpu.VMEM_SHARED((PR, D), jnp.float32),
        pltpu.VMEM((PKP, PCH), jnp.int32),
        pltpu.VMEM((PCH, D), jnp.float32),
        pltpu.VMEM((PCH, D), jnp.float32),
    ],
)


def _layer_body(h_ref, agg_ref, slW_ref, slb_ref, W1_ref, b1_ref, g_ref,
                be_ref, W2_ref, b2_ref, out_ref):
    h = h_ref[...]
    acc = jnp.dot(h, slW_ref[...], preferred_element_type=jnp.float32)
    acc = acc + slb_ref[...]
    for r in range(R):
        t = h + agg_ref[r]
        u = jnp.dot(t, W1_ref[r], preferred_element_type=jnp.float32)
        u = (u + b1_ref[r]) * (g_ref[r] * _BN_INV) + be_ref[r]
        u = jnp.maximum(u, 0.0)
        acc = acc + jnp.dot(u, W2_ref[r], preferred_element_type=jnp.float32)
        acc = acc + b2_ref[r]
    out_ref[...] = jnp.maximum(acc, 0.0)


_layer = pl.pallas_call(
    _layer_body,
    grid=(N // BL,),
    in_specs=[
        pl.BlockSpec((BL, D), lambda i: (i, 0)),
        pl.BlockSpec((R, BL, D), lambda i: (0, i, 0)),
        pl.BlockSpec((D, D), lambda i: (0, 0)),
        pl.BlockSpec((1, D), lambda i: (0, 0)),
        pl.BlockSpec((R, D, D), lambda i: (0, 0, 0)),
        pl.BlockSpec((R, 1, D), lambda i: (0, 0, 0)),
        pl.BlockSpec((R, 1, D), lambda i: (0, 0, 0)),
        pl.BlockSpec((R, 1, D), lambda i: (0, 0, 0)),
        pl.BlockSpec((R, D, D), lambda i: (0, 0, 0)),
        pl.BlockSpec((R, 1, D), lambda i: (0, 0, 0)),
    ],
    out_specs=pl.BlockSpec((BL, D), lambda i: (i, 0)),
    out_shape=jax.ShapeDtypeStruct((N, D), jnp.float32),
)


def _head_body(sums_ref, cnt_ref, w_ref, b_ref, out_ref):
    s = sums_ref[0, :G, :] + sums_ref[1, :G, :]
    c = cnt_ref[0, :G, :1] + cnt_ref[1, :G, :1]
    pooled = s / jnp.maximum(c, 1.0)
    out_ref[...] = (jnp.dot(pooled, w_ref[...],
                            preferred_element_type=jnp.float32) + b_ref[...])


_head = pl.pallas_call(
    _head_body,
    in_specs=[
        pl.BlockSpec((NC, PR, D), lambda: (0, 0, 0)),
        pl.BlockSpec((NC, PR, D), lambda: (0, 0, 0)),
        pl.BlockSpec((D, T), lambda: (0, 0)),
        pl.BlockSpec((1, T), lambda: (0, 0)),
    ],
    out_specs=pl.BlockSpec((G, T), lambda: (0, 0)),
    out_shape=jax.ShapeDtypeStruct((G, T), jnp.float32),
)


def kernel(x, edge_index, edge_type, batch,
           c1_slW, c1_slb, c1_W1, c1_b1, c1_g, c1_be, c1_W2, c1_b2,
           c2_slW, c2_slb, c2_W1, c2_b1, c2_g, c2_be, c2_W2, c2_b2,
           c3_slW, c3_slb, c3_W1, c3_b1, c3_g, c3_be, c3_W2, c3_b2,
           lin_W, lin_b):
    src = edge_index[0]
    dst = edge_index[1]
    packed = jnp.left_shift(src, SRC_SHIFT) | (edge_type * NPAD + dst)
    packed = jnp.concatenate(
        [packed, jnp.full((NS * EPT - E,), DUMMY_DST, jnp.int32)])
    zrow = jnp.zeros((ZR, D), jnp.float32)

    h = x
    for (slW, slb, W1, b1, g, be, W2, b2) in (
            (c1_slW, c1_slb, c1_W1, c1_b1, c1_g, c1_be, c1_W2, c1_b2),
            (c2_slW, c2_slb, c2_W1, c2_b1, c2_g, c2_be, c2_W2, c2_b2),
            (c3_slW, c3_slb, c3_W1, c3_b1, c3_g, c3_be, c3_W2, c3_b2)):
        agg4 = _agg(h, packed, zrow)
        h = _layer(h, agg4, slW, slb.reshape(1, D), W1,
                   b1.reshape(R, 1, D), g.reshape(R, 1, D),
                   be.reshape(R, 1, D), W2, b2.reshape(R, 1, D))

    h_pad = jnp.concatenate([h, jnp.zeros((NPOOL - N, D), jnp.float32)])
    ids_p = jnp.concatenate(
        [batch, jnp.full((NPOOL - N,), G, jnp.int32)]).reshape(
            NC * NS, PK, PCH)
    ids_p = jnp.pad(ids_p, ((0, 0), (0, PKP - PK), (0, 0)),
                    constant_values=G)
    ones = jnp.ones((PCH, D), jnp.float32)
    zrow_pool = jnp.zeros((PZR, D), jnp.float32)
    sums, cnt = _pool(h_pad, ids_p, ones, zrow_pool)
    return _head(sums, cnt, lin_W, lin_b.reshape(1, T))


# speedup vs baseline: 52.1415x; 2.3641x over previous
import math

import jax
import jax.numpy as jnp
from jax import lax
from jax.experimental import pallas as pl
from jax.experimental.pallas import tpu as pltpu
from jax.experimental.pallas import tpu_sc as plsc

N, E, D, R, T, G = 10000, 320000, 128, 4, 12, 256
BN_EPS = 1e-5
_BN_INV = 1.0 / math.sqrt(1.0 + BN_EPS)

NC, NS = 2, 16
NPAD = 10240
ZR = NPAD // NS
UCH = 64
UNITS = 320
EPT = UNITS * UCH
DUMMY_DST = N
TRASH = NPAD - 64
SRC_SHIFT = 16

PK = 3
PKP = 8
PCH = 128
NPOOL = NC * NS * PK * PCH
PR = 384
PZR = PR // NS

BL = 400


def _agg_body(h_hbm, packed_hbm, zrow_hbm, out_hbm,
              seg_sp, packed_v, sidxA, didxA, sidxB, didxB,
              rowsA, rowsB, semA, semB):
    cid = lax.axis_index("c")
    sid = lax.axis_index("s")
    pltpu.sync_copy(packed_hbm.at[pl.ds(sid * EPT, EPT)], packed_v)

    for p in range(2):
        for cc in range(NC):
            rr = 2 * p + cc

            @pl.when(cid == cc)
            def _(rr=rr):
                pltpu.sync_copy(zrow_hbm, seg_sp.at[pl.ds(sid * ZR, ZR)])
                plsc.subcore_barrier()

                def compute(u, sx, dx, rr=rr):
                    iota = lax.iota(jnp.int32, 16)
                    for v in range(UCH // 16):
                        w = packed_v[pl.ds(u * UCH + v * 16, 16)]
                        off = (w & 0xFFFF) - rr * NPAD
                        ok = (off >= 0) & (off < NPAD)
                        sx[0, pl.ds(v * 16, 16)] = lax.shift_right_logical(
                            w, SRC_SHIFT)
                        dx[0, pl.ds(v * 16, 16)] = jnp.where(
                            ok, off, TRASH + v * 16 + iota)

                compute(0, sidxA, didxA)
                pltpu.async_copy(h_hbm.at[sidxA.at[0]], rowsA, semA)

                def unit_body(j, carry):
                    compute(2 * j + 1, sidxB, didxB)
                    pltpu.make_async_copy(
                        h_hbm.at[sidxA.at[0]], rowsA, semA).wait()
                    pltpu.async_copy(h_hbm.at[sidxB.at[0]], rowsB, semB)
                    pltpu.sync_copy(rowsA, seg_sp.at[didxA.at[0]], add=True)
                    compute(2 * j + 2, sidxA, didxA)
                    pltpu.make_async_copy(
                        h_hbm.at[sidxB.at[0]], rowsB, semB).wait()
                    pltpu.async_copy(h_hbm.at[sidxA.at[0]], rowsA, semA)
                    pltpu.sync_copy(rowsB, seg_sp.at[didxB.at[0]], add=True)
                    return carry

                lax.fori_loop(0, UNITS // 2 - 1, unit_body, 0)
                compute(UNITS - 1, sidxB, didxB)
                pltpu.make_async_copy(
                    h_hbm.at[sidxA.at[0]], rowsA, semA).wait()
                pltpu.async_copy(h_hbm.at[sidxB.at[0]], rowsB, semB)
                pltpu.sync_copy(rowsA, seg_sp.at[didxA.at[0]], add=True)
                pltpu.make_async_copy(
                    h_hbm.at[sidxB.at[0]], rowsB, semB).wait()
                pltpu.sync_copy(rowsB, seg_sp.at[didxB.at[0]], add=True)

                plsc.subcore_barrier()
                pltpu.sync_copy(seg_sp.at[pl.ds(sid * ZR, ZR)],
                                out_hbm.at[rr].at[pl.ds(sid * ZR, ZR)])
                plsc.subcore_barrier()


_sc_mesh = plsc.VectorSubcoreMesh(
    core_axis_name="c", subcore_axis_name="s", num_cores=NC, num_subcores=NS)

_agg = pl.kernel(
    _agg_body,
    out_type=jax.ShapeDtypeStruct((R, NPAD, D), jnp.float32),
    mesh=_sc_mesh,
    scratch_types=[
        pltpu.VMEM_SHARED((NPAD, D), jnp.float32),
        pltpu.VMEM((EPT,), jnp.int32),
        pltpu.VMEM((1, UCH), jnp.int32),
        pltpu.VMEM((1, UCH), jnp.int32),
        pltpu.VMEM((1, UCH), jnp.int32),
        pltpu.VMEM((1, UCH), jnp.int32),
        pltpu.VMEM((UCH, D), jnp.float32),
        pltpu.VMEM((UCH, D), jnp.float32),
        pltpu.SemaphoreType.DMA,
        pltpu.SemaphoreType.DMA,
    ],
)


def _pool_body(h_hbm, ids_hbm, ones_hbm, zrow_hbm, sums_hbm, cnt_hbm,
               pool_sp, cnt_sp, ids_v, rows_v, ones_v):
    cid = lax.axis_index("c")
    sid = lax.axis_index("s")
    wid = cid * NS + sid
    pltpu.sync_copy(ids_hbm.at[wid], ids_v)
    pltpu.sync_copy(ones_hbm, ones_v)
    pltpu.sync_copy(zrow_hbm, pool_sp.at[pl.ds(sid * PZR, PZR)])
    pltpu.sync_copy(zrow_hbm, cnt_sp.at[pl.ds(sid * PZR, PZR)])
    plsc.subcore_barrier()
    base = wid * PK * PCH
    for k in range(PK):
        pltpu.sync_copy(h_hbm.at[pl.ds(base + k * PCH, PCH)], rows_v)
        pltpu.sync_copy(rows_v, pool_sp.at[ids_v.at[k]], add=True)
        pltpu.sync_copy(ones_v, cnt_sp.at[ids_v.at[k]], add=True)
    plsc.subcore_barrier()
    pltpu.sync_copy(pool_sp.at[pl.ds(sid * PZR, PZR)],
                    sums_hbm.at[cid].at[pl.ds(sid * PZR, PZR)])
    pltpu.sync_copy(cnt_sp.at[pl.ds(sid * PZR, PZR)],
                    cnt_hbm.at[cid].at[pl.ds(sid * PZR, PZR)])


_pool = pl.kernel(
    _pool_body,
    out_type=(jax.ShapeDtypeStruct((NC, PR, D), jnp.float32),
              jax.ShapeDtypeStruct((NC, PR, D), jnp.float32)),
    mesh=_sc_mesh,
    scratch_types=[
        pltpu.VMEM_SHARED((PR, D), jnp.float32),
        pltpu.VMEM_SHARED((PR, D), jnp.float32),
        pltpu.VMEM((PKP, PCH), jnp.int32),
        pltpu.VMEM((PCH, D), jnp.float32),
        pltpu.VMEM((PCH, D), jnp.float32),
    ],
)


def _layer_body(h_ref, agg_ref, slW_ref, slb_ref, W1_ref, b1_ref, g_ref,
                be_ref, W2_ref, b2_ref, out_ref):
    h = h_ref[...]
    acc = jnp.dot(h, slW_ref[...], preferred_element_type=jnp.float32)
    acc = acc + slb_ref[...]
    for r in range(R):
        t = h + agg_ref[r]
        u = jnp.dot(t, W1_ref[r], preferred_element_type=jnp.float32)
        u = (u + b1_ref[r]) * (g_ref[r] * _BN_INV) + be_ref[r]
        u = jnp.maximum(u, 0.0)
        acc = acc + jnp.dot(u, W2_ref[r], preferred_element_type=jnp.float32)
        acc = acc + b2_ref[r]
    out_ref[...] = jnp.maximum(acc, 0.0)


_layer = pl.pallas_call(
    _layer_body,
    grid=(N // BL,),
    in_specs=[
        pl.BlockSpec((BL, D), lambda i: (i, 0)),
        pl.BlockSpec((R, BL, D), lambda i: (0, i, 0)),
        pl.BlockSpec((D, D), lambda i: (0, 0)),
        pl.BlockSpec((1, D), lambda i: (0, 0)),
        pl.BlockSpec((R, D, D), lambda i: (0, 0, 0)),
        pl.BlockSpec((R, 1, D), lambda i: (0, 0, 0)),
        pl.BlockSpec((R, 1, D), lambda i: (0, 0, 0)),
        pl.BlockSpec((R, 1, D), lambda i: (0, 0, 0)),
        pl.BlockSpec((R, D, D), lambda i: (0, 0, 0)),
        pl.BlockSpec((R, 1, D), lambda i: (0, 0, 0)),
    ],
    out_specs=pl.BlockSpec((BL, D), lambda i: (i, 0)),
    out_shape=jax.ShapeDtypeStruct((N, D), jnp.float32),
)


def _head_body(sums_ref, cnt_ref, w_ref, b_ref, out_ref):
    s = sums_ref[0, :G, :] + sums_ref[1, :G, :]
    c = cnt_ref[0, :G, :1] + cnt_ref[1, :G, :1]
    pooled = s / jnp.maximum(c, 1.0)
    out_ref[...] = (jnp.dot(pooled, w_ref[...],
                            preferred_element_type=jnp.float32) + b_ref[...])


_head = pl.pallas_call(
    _head_body,
    in_specs=[
        pl.BlockSpec((NC, PR, D), lambda: (0, 0, 0)),
        pl.BlockSpec((NC, PR, D), lambda: (0, 0, 0)),
        pl.BlockSpec((D, T), lambda: (0, 0)),
        pl.BlockSpec((1, T), lambda: (0, 0)),
    ],
    out_specs=pl.BlockSpec((G, T), lambda: (0, 0)),
    out_shape=jax.ShapeDtypeStruct((G, T), jnp.float32),
)


def kernel(x, edge_index, edge_type, batch,
           c1_slW, c1_slb, c1_W1, c1_b1, c1_g, c1_be, c1_W2, c1_b2,
           c2_slW, c2_slb, c2_W1, c2_b1, c2_g, c2_be, c2_W2, c2_b2,
           c3_slW, c3_slb, c3_W1, c3_b1, c3_g, c3_be, c3_W2, c3_b2,
           lin_W, lin_b):
    src = edge_index[0]
    dst = edge_index[1]
    packed = jnp.left_shift(src, SRC_SHIFT) | (edge_type * NPAD + dst)
    pad_src = jnp.arange(NS * EPT - E, dtype=jnp.int32) % N
    packed = jnp.concatenate(
        [packed, jnp.left_shift(pad_src, SRC_SHIFT) | DUMMY_DST])
    zrow = jnp.zeros((ZR, D), jnp.float32)

    h = x
    for (slW, slb, W1, b1, g, be, W2, b2) in (
            (c1_slW, c1_slb, c1_W1, c1_b1, c1_g, c1_be, c1_W2, c1_b2),
            (c2_slW, c2_slb, c2_W1, c2_b1, c2_g, c2_be, c2_W2, c2_b2),
            (c3_slW, c3_slb, c3_W1, c3_b1, c3_g, c3_be, c3_W2, c3_b2)):
        agg4 = _agg(h, packed, zrow)
        h = _layer(h, agg4, slW, slb.reshape(1, D), W1,
                   b1.reshape(R, 1, D), g.reshape(R, 1, D),
                   be.reshape(R, 1, D), W2, b2.reshape(R, 1, D))

    h_pad = jnp.concatenate([h, jnp.zeros((NPOOL - N, D), jnp.float32)])
    ids_p = jnp.concatenate(
        [batch, jnp.full((NPOOL - N,), G, jnp.int32)]).reshape(
            NC * NS, PK, PCH)
    ids_p = jnp.pad(ids_p, ((0, 0), (0, PKP - PK), (0, 0)),
                    constant_values=G)
    ones = jnp.ones((PCH, D), jnp.float32)
    zrow_pool = jnp.zeros((PZR, D), jnp.float32)
    sums, cnt = _pool(h_pad, ids_p, ones, zrow_pool)
    return _head(sums, cnt, lin_W, lin_b.reshape(1, T))


# speedup vs baseline: 58.3455x; 1.1190x over previous
import math

import jax
import jax.numpy as jnp
from jax import lax
from jax.experimental import pallas as pl
from jax.experimental.pallas import tpu as pltpu
from jax.experimental.pallas import tpu_sc as plsc

N, E, D, R, T, G = 10000, 320000, 128, 4, 12, 256
BN_EPS = 1e-5
_BN_INV = 1.0 / math.sqrt(1.0 + BN_EPS)

NC, NS = 2, 16
NPAD = 10240
ZR = NPAD // NS
UCH = 80
UNITS = 256
EPT = UNITS * UCH
DUMMY_DST = N
TRASH = NPAD - 64
SRC_SHIFT = 16

PK = 3
PKP = 8
PCH = 128
NPOOL = NC * NS * PK * PCH
PR = 384
PZR = PR // NS

BL = 400


def _agg_body(h_hbm, packed_hbm, zrow_hbm, out_hbm,
              seg_sp, packed_v, sidxA, didxA, sidxB, didxB,
              rowsA, rowsB, semA, semB):
    cid = lax.axis_index("c")
    sid = lax.axis_index("s")
    pltpu.sync_copy(packed_hbm.at[pl.ds(sid * EPT, EPT)], packed_v)

    for p in range(2):
        for cc in range(NC):
            rr = 2 * p + cc

            @pl.when(cid == cc)
            def _(rr=rr):
                pltpu.sync_copy(zrow_hbm, seg_sp.at[pl.ds(sid * ZR, ZR)])
                plsc.subcore_barrier()

                def compute(u, sx, dx, rr=rr):
                    iota = lax.iota(jnp.int32, 16)
                    for v in range(UCH // 16):
                        w = packed_v[pl.ds(u * UCH + v * 16, 16)]
                        off = (w & 0xFFFF) - rr * NPAD
                        ok = (off >= 0) & (off < NPAD)
                        sx[0, pl.ds(v * 16, 16)] = lax.shift_right_logical(
                            w, SRC_SHIFT)
                        dx[0, pl.ds(v * 16, 16)] = jnp.where(
                            ok, off, TRASH + v * 16 + iota)

                compute(0, sidxA, didxA)
                pltpu.async_copy(h_hbm.at[sidxA.at[0]], rowsA, semA)

                def unit_body(j, carry):
                    compute(2 * j + 1, sidxB, didxB)
                    pltpu.make_async_copy(
                        h_hbm.at[sidxA.at[0]], rowsA, semA).wait()
                    pltpu.async_copy(h_hbm.at[sidxB.at[0]], rowsB, semB)
                    pltpu.sync_copy(rowsA, seg_sp.at[didxA.at[0]], add=True)
                    compute(2 * j + 2, sidxA, didxA)
                    pltpu.make_async_copy(
                        h_hbm.at[sidxB.at[0]], rowsB, semB).wait()
                    pltpu.async_copy(h_hbm.at[sidxA.at[0]], rowsA, semA)
                    pltpu.sync_copy(rowsB, seg_sp.at[didxB.at[0]], add=True)
                    return carry

                lax.fori_loop(0, UNITS // 2 - 1, unit_body, 0)
                compute(UNITS - 1, sidxB, didxB)
                pltpu.make_async_copy(
                    h_hbm.at[sidxA.at[0]], rowsA, semA).wait()
                pltpu.async_copy(h_hbm.at[sidxB.at[0]], rowsB, semB)
                pltpu.sync_copy(rowsA, seg_sp.at[didxA.at[0]], add=True)
                pltpu.make_async_copy(
                    h_hbm.at[sidxB.at[0]], rowsB, semB).wait()
                pltpu.sync_copy(rowsB, seg_sp.at[didxB.at[0]], add=True)

                plsc.subcore_barrier()
                pltpu.sync_copy(seg_sp.at[pl.ds(sid * ZR, ZR)],
                                out_hbm.at[rr].at[pl.ds(sid * ZR, ZR)])
                plsc.subcore_barrier()


_sc_mesh = plsc.VectorSubcoreMesh(
    core_axis_name="c", subcore_axis_name="s", num_cores=NC, num_subcores=NS)

_agg = pl.kernel(
    _agg_body,
    out_type=jax.ShapeDtypeStruct((R, NPAD, D), jnp.float32),
    mesh=_sc_mesh,
    scratch_types=[
        pltpu.VMEM_SHARED((NPAD, D), jnp.float32),
        pltpu.VMEM((EPT,), jnp.int32),
        pltpu.VMEM((1, UCH), jnp.int32),
        pltpu.VMEM((1, UCH), jnp.int32),
        pltpu.VMEM((1, UCH), jnp.int32),
        pltpu.VMEM((1, UCH), jnp.int32),
        pltpu.VMEM((UCH, D), jnp.float32),
        pltpu.VMEM((UCH, D), jnp.float32),
        pltpu.SemaphoreType.DMA,
        pltpu.SemaphoreType.DMA,
    ],
)


def _pool_body(h_hbm, ids_hbm, ones_hbm, zrow_hbm, sums_hbm, cnt_hbm,
               pool_sp, cnt_sp, ids_v, rows_v, ones_v):
    cid = lax.axis_index("c")
    sid = lax.axis_index("s")
    wid = cid * NS + sid
    pltpu.sync_copy(ids_hbm.at[wid], ids_v)
    pltpu.sync_copy(ones_hbm, ones_v)
    pltpu.sync_copy(zrow_hbm, pool_sp.at[pl.ds(sid * PZR, PZR)])
    pltpu.sync_copy(zrow_hbm, cnt_sp.at[pl.ds(sid * PZR, PZR)])
    plsc.subcore_barrier()
    base = wid * PK * PCH
    for k in range(PK):
        pltpu.sync_copy(h_hbm.at[pl.ds(base + k * PCH, PCH)], rows_v)
        pltpu.sync_copy(rows_v, pool_sp.at[ids_v.at[k]], add=True)
        pltpu.sync_copy(ones_v, cnt_sp.at[ids_v.at[k]], add=True)
    plsc.subcore_barrier()
    pltpu.sync_copy(pool_sp.at[pl.ds(sid * PZR, PZR)],
                    sums_hbm.at[cid].at[pl.ds(sid * PZR, PZR)])
    pltpu.sync_copy(cnt_sp.at[pl.ds(sid * PZR, PZR)],
                    cnt_hbm.at[cid].at[pl.ds(sid * PZR, PZR)])


_pool = pl.kernel(
    _pool_body,
    out_type=(jax.ShapeDtypeStruct((NC, PR, D), jnp.float32),
              jax.ShapeDtypeStruct((NC, PR, D), jnp.float32)),
    mesh=_sc_mesh,
    scratch_types=[
        pltpu.VMEM_SHARED((PR, D), jnp.float32),
        pltpu.VMEM_SHARED((PR, D), jnp.float32),
        pltpu.VMEM((PKP, PCH), jnp.int32),
        pltpu.VMEM((PCH, D), jnp.float32),
        pltpu.VMEM((PCH, D), jnp.float32),
    ],
)


def _layer_body(h_ref, agg_ref, slW_ref, slb_ref, W1_ref, b1_ref, g_ref,
                be_ref, W2_ref, b2_ref, out_ref):
    h = h_ref[...]
    acc = jnp.dot(h, slW_ref[...], preferred_element_type=jnp.float32)
    acc = acc + slb_ref[...]
    for r in range(R):
        t = h + agg_ref[r]
        u = jnp.dot(t, W1_ref[r], preferred_element_type=jnp.float32)
        u = (u + b1_ref[r]) * (g_ref[r] * _BN_INV) + be_ref[r]
        u = jnp.maximum(u, 0.0)
        acc = acc + jnp.dot(u, W2_ref[r], preferred_element_type=jnp.float32)
        acc = acc + b2_ref[r]
    out_ref[...] = jnp.maximum(acc, 0.0)


_layer = pl.pallas_call(
    _layer_body,
    grid=(N // BL,),
    in_specs=[
        pl.BlockSpec((BL, D), lambda i: (i, 0)),
        pl.BlockSpec((R, BL, D), lambda i: (0, i, 0)),
        pl.BlockSpec((D, D), lambda i: (0, 0)),
        pl.BlockSpec((1, D), lambda i: (0, 0)),
        pl.BlockSpec((R, D, D), lambda i: (0, 0, 0)),
        pl.BlockSpec((R, 1, D), lambda i: (0, 0, 0)),
        pl.BlockSpec((R, 1, D), lambda i: (0, 0, 0)),
        pl.BlockSpec((R, 1, D), lambda i: (0, 0, 0)),
        pl.BlockSpec((R, D, D), lambda i: (0, 0, 0)),
        pl.BlockSpec((R, 1, D), lambda i: (0, 0, 0)),
    ],
    out_specs=pl.BlockSpec((BL, D), lambda i: (i, 0)),
    out_shape=jax.ShapeDtypeStruct((N, D), jnp.float32),
)


def _head_body(sums_ref, cnt_ref, w_ref, b_ref, out_ref):
    s = sums_ref[0, :G, :] + sums_ref[1, :G, :]
    c = cnt_ref[0, :G, :1] + cnt_ref[1, :G, :1]
    pooled = s / jnp.maximum(c, 1.0)
    out_ref[...] = (jnp.dot(pooled, w_ref[...],
                            preferred_element_type=jnp.float32) + b_ref[...])


_head = pl.pallas_call(
    _head_body,
    in_specs=[
        pl.BlockSpec((NC, PR, D), lambda: (0, 0, 0)),
        pl.BlockSpec((NC, PR, D), lambda: (0, 0, 0)),
        pl.BlockSpec((D, T), lambda: (0, 0)),
        pl.BlockSpec((1, T), lambda: (0, 0)),
    ],
    out_specs=pl.BlockSpec((G, T), lambda: (0, 0)),
    out_shape=jax.ShapeDtypeStruct((G, T), jnp.float32),
)


def kernel(x, edge_index, edge_type, batch,
           c1_slW, c1_slb, c1_W1, c1_b1, c1_g, c1_be, c1_W2, c1_b2,
           c2_slW, c2_slb, c2_W1, c2_b1, c2_g, c2_be, c2_W2, c2_b2,
           c3_slW, c3_slb, c3_W1, c3_b1, c3_g, c3_be, c3_W2, c3_b2,
           lin_W, lin_b):
    src = edge_index[0]
    dst = edge_index[1]
    packed = jnp.left_shift(src, SRC_SHIFT) | (edge_type * NPAD + dst)
    pad_src = jnp.arange(NS * EPT - E, dtype=jnp.int32) % N
    packed = jnp.concatenate(
        [packed, jnp.left_shift(pad_src, SRC_SHIFT) | DUMMY_DST])
    zrow = jnp.zeros((ZR, D), jnp.float32)

    h = x
    for (slW, slb, W1, b1, g, be, W2, b2) in (
            (c1_slW, c1_slb, c1_W1, c1_b1, c1_g, c1_be, c1_W2, c1_b2),
            (c2_slW, c2_slb, c2_W1, c2_b1, c2_g, c2_be, c2_W2, c2_b2),
            (c3_slW, c3_slb, c3_W1, c3_b1, c3_g, c3_be, c3_W2, c3_b2)):
        agg4 = _agg(h, packed, zrow)
        h = _layer(h, agg4, slW, slb.reshape(1, D), W1,
                   b1.reshape(R, 1, D), g.reshape(R, 1, D),
                   be.reshape(R, 1, D), W2, b2.reshape(R, 1, D))

    h_pad = jnp.concatenate([h, jnp.zeros((NPOOL - N, D), jnp.float32)])
    ids_p = jnp.concatenate(
        [batch, jnp.full((NPOOL - N,), G, jnp.int32)]).reshape(
            NC * NS, PK, PCH)
    ids_p = jnp.pad(ids_p, ((0, 0), (0, PKP - PK), (0, 0)),
                    constant_values=G)
    ones = jnp.ones((PCH, D), jnp.float32)
    zrow_pool = jnp.zeros((PZR, D), jnp.float32)
    sums, cnt = _pool(h_pad, ids_p, ones, zrow_pool)
    return _head(sums, cnt, lin_W, lin_b.reshape(1, T))
